# Initial kernel scaffold; baseline (speedup 1.0000x reference)
import functools

import jax
import jax.numpy as jnp
from jax import lax
from jax.experimental import pallas as pl
from jax.experimental.pallas import tpu as pltpu
from jax.experimental.pallas import tpu_sc as plsc

N = 10000
E = 320000
D_IN = 128
D_H = 16
D_OUT = 64

NC = 2
NS = 16
NW = NC * NS
EPT = E // NW
CB = 80
NCHUNK = EPT // CB
RPS = N // NS
ZR = 125

_MESH = plsc.VectorSubcoreMesh(
    core_axis_name="c", subcore_axis_name="s", num_cores=NC, num_subcores=NS)


def _make_deg():
  scratch = [
      pltpu.VMEM((NCHUNK, CB), jnp.int32),
      pltpu.VMEM((ZR, D_H), jnp.float32),
      pltpu.VMEM((CB, D_H), jnp.float32),
      pltpu.VMEM_SHARED((N, D_H), jnp.float32),
  ]

  @functools.partial(
      pl.kernel,
      out_type=jax.ShapeDtypeStruct((NC, N, D_H), jnp.float32),
      mesh=_MESH,
      scratch_types=scratch,
  )
  def deg_kernel(ei_hbm, out_hbm, dst_v, zbuf, obuf, acc):
    c = lax.axis_index("c")
    s = lax.axis_index("s")
    wid = c * NS + s
    pltpu.sync_copy(ei_hbm.at[1, wid], dst_v)

    def fill_z(r, _):
      zbuf[r, :] = jnp.zeros((D_H,), jnp.float32)
      return 0
    lax.fori_loop(0, ZR, fill_z, 0)

    def fill_o(r, _):
      obuf[r, :] = jnp.ones((D_H,), jnp.float32)
      return 0
    lax.fori_loop(0, CB, fill_o, 0)

    for k in range(RPS // ZR):
      pltpu.sync_copy(zbuf, acc.at[pl.ds(s * RPS + k * ZR, ZR)])
    plsc.subcore_barrier()

    def chunk(i, _):
      pltpu.sync_copy(obuf, acc.at[dst_v.at[i]], add=True)
      return 0
    lax.fori_loop(0, NCHUNK, chunk, 0)

    plsc.subcore_barrier()
    pltpu.sync_copy(acc.at[pl.ds(s * RPS, RPS)],
                    out_hbm.at[c, pl.ds(s * RPS, RPS)])

  return deg_kernel


def _make_agg(d):
  scratch = [
      pltpu.VMEM((NCHUNK, CB), jnp.int32),
      pltpu.VMEM((NCHUNK, CB), jnp.int32),
      pltpu.VMEM((CB, d), jnp.float32),
      pltpu.VMEM((ZR, d), jnp.float32),
      pltpu.VMEM_SHARED((N, d), jnp.float32),
      pltpu.SemaphoreType.DMA,
  ]

  @functools.partial(
      pl.kernel,
      out_type=jax.ShapeDtypeStruct((NC, N, d), jnp.float32),
      mesh=_MESH,
      scratch_types=scratch,
  )
  def agg_kernel(y_hbm, ei_hbm, out_hbm, src_v, dst_v, buf, zbuf, acc, sem):
    c = lax.axis_index("c")
    s = lax.axis_index("s")
    wid = c * NS + s
    pltpu.sync_copy(ei_hbm.at[0, wid], src_v)
    pltpu.sync_copy(ei_hbm.at[1, wid], dst_v)

    def fill_z(r, _):
      for j in range(d // 16):
        zbuf[r, pl.ds(j * 16, 16)] = jnp.zeros((16,), jnp.float32)
      return 0
    lax.fori_loop(0, ZR, fill_z, 0)

    for k in range(RPS // ZR):
      pltpu.sync_copy(zbuf, acc.at[pl.ds(s * RPS + k * ZR, ZR)])
    plsc.subcore_barrier()

    def chunk(i, _):
      pltpu.async_copy(y_hbm.at[src_v.at[i]], buf, sem).wait()
      pltpu.sync_copy(buf, acc.at[dst_v.at[i]], add=True)
      return 0
    lax.fori_loop(0, NCHUNK, chunk, 0)

    plsc.subcore_barrier()
    pltpu.sync_copy(acc.at[pl.ds(s * RPS, RPS)],
                    out_hbm.at[c, pl.ds(s * RPS, RPS)])

  return agg_kernel


_deg_call = _make_deg()
_agg16_call = _make_agg(D_H)
_agg64_call = _make_agg(D_OUT)


BN = 2000


def _layer1_body(x_ref, w1_ref, dg_ref, y1_ref, dinv_ref):
  deg = dg_ref[0] + dg_ref[1] + 1.0
  dinv = lax.rsqrt(deg)
  xw = jnp.dot(x_ref[...], w1_ref[...], preferred_element_type=jnp.float32)
  y1_ref[...] = xw * dinv
  dinv_ref[...] = dinv


def _layer1_call(x, w1, degp):
  return pl.pallas_call(
      _layer1_body,
      grid=(N // BN,),
      in_specs=[
          pl.BlockSpec((BN, D_IN), lambda i: (i, 0)),
          pl.BlockSpec((D_IN, D_H), lambda i: (0, 0)),
          pl.BlockSpec((NC, BN, D_H), lambda i: (0, i, 0)),
      ],
      out_specs=[
          pl.BlockSpec((BN, D_H), lambda i: (i, 0)),
          pl.BlockSpec((BN, D_H), lambda i: (i, 0)),
      ],
      out_shape=[
          jax.ShapeDtypeStruct((N, D_H), jnp.float32),
          jax.ShapeDtypeStruct((N, D_H), jnp.float32),
      ],
  )(x, w1, degp)


def _layer2_body(ag_ref, y1_ref, dinv_ref, b1_ref, w2_ref, y2_ref):
  t = ag_ref[0] + ag_ref[1] + y1_ref[...]
  h = jnp.maximum(dinv_ref[...] * t + b1_ref[...], 0.0)
  hw = jnp.dot(h, w2_ref[...], preferred_element_type=jnp.float32)
  dinv64 = lax.broadcast_in_dim(dinv_ref[...][:, 0:1], (BN, D_OUT), (0, 1))
  y2_ref[...] = hw * dinv64


def _layer2_call(agg1, y1, dinv, b1r, w2):
  return pl.pallas_call(
      _layer2_body,
      grid=(N // BN,),
      in_specs=[
          pl.BlockSpec((NC, BN, D_H), lambda i: (0, i, 0)),
          pl.BlockSpec((BN, D_H), lambda i: (i, 0)),
          pl.BlockSpec((BN, D_H), lambda i: (i, 0)),
          pl.BlockSpec((1, D_H), lambda i: (0, 0)),
          pl.BlockSpec((D_H, D_OUT), lambda i: (0, 0)),
      ],
      out_specs=pl.BlockSpec((BN, D_OUT), lambda i: (i, 0)),
      out_shape=jax.ShapeDtypeStruct((N, D_OUT), jnp.float32),
  )(agg1, y1, dinv, b1r, w2)


def _out_body(ag_ref, y2_ref, dinv_ref, b2_ref, o_ref):
  dinv64 = lax.broadcast_in_dim(dinv_ref[...][:, 0:1], (BN, D_OUT), (0, 1))
  o = dinv64 * (ag_ref[0] + ag_ref[1] + y2_ref[...]) + b2_ref[...]
  m = jnp.max(o, axis=1, keepdims=True)
  lse = jnp.log(jnp.sum(jnp.exp(o - m), axis=1, keepdims=True)) + m
  o_ref[...] = o - lse


def _out_call(agg2, y2, dinv, b2r):
  return pl.pallas_call(
      _out_body,
      grid=(N // BN,),
      in_specs=[
          pl.BlockSpec((NC, BN, D_OUT), lambda i: (0, i, 0)),
          pl.BlockSpec((BN, D_OUT), lambda i: (i, 0)),
          pl.BlockSpec((BN, D_H), lambda i: (i, 0)),
          pl.BlockSpec((1, D_OUT), lambda i: (0, 0)),
      ],
      out_specs=pl.BlockSpec((BN, D_OUT), lambda i: (i, 0)),
      out_shape=jax.ShapeDtypeStruct((N, D_OUT), jnp.float32),
  )(agg2, y2, dinv, b2r)


def kernel(x, edge_index, W1, b1, W2, b2):
  ei = edge_index.astype(jnp.int32).reshape(2, NW, NCHUNK, CB)
  degp = _deg_call(ei)
  y1, dinv = _layer1_call(x, W1, degp)
  agg1 = _agg16_call(y1, ei)
  y2 = _layer2_call(agg1, y1, dinv, b1.reshape(1, D_H), W2)
  agg2 = _agg64_call(y2, ei)
  return _out_call(agg2, y2, dinv, b2.reshape(1, D_OUT))


# speedup vs baseline: 27.3494x; 27.3494x over previous
import functools

import jax
import jax.numpy as jnp
from jax import lax
from jax.experimental import pallas as pl
from jax.experimental.pallas import tpu as pltpu
from jax.experimental.pallas import tpu_sc as plsc

N = 10000
E = 320000
D_IN = 128
D_H = 16
D_OUT = 64

NC = 2
NS = 16
NW = NC * NS
EPT = E // NW
CB = 80
NCHUNK = EPT // CB
ACC_N = 10240
RPS = ACC_N // NS
ZR = 128

_MESH = plsc.VectorSubcoreMesh(
    core_axis_name="c", subcore_axis_name="s", num_cores=NC, num_subcores=NS)


def _make_deg():
  scratch = [
      pltpu.VMEM((NCHUNK, CB), jnp.int32),
      pltpu.VMEM((ZR, D_H), jnp.float32),
      pltpu.VMEM((CB, D_H), jnp.float32),
      pltpu.VMEM_SHARED((ACC_N, D_H), jnp.float32),
  ]

  @functools.partial(
      pl.kernel,
      out_type=jax.ShapeDtypeStruct((NC, ACC_N, D_H), jnp.float32),
      mesh=_MESH,
      scratch_types=scratch,
      compiler_params=pltpu.CompilerParams(use_tc_tiling_on_sc=False),
  )
  def deg_kernel(ei_hbm, out_hbm, dst_v, zbuf, obuf, acc):
    c = lax.axis_index("c")
    s = lax.axis_index("s")
    wid = c * NS + s
    pltpu.sync_copy(ei_hbm.at[1, wid], dst_v)

    def fill_z(r, _):
      zbuf[r, :] = jnp.zeros((D_H,), jnp.float32)
      return 0
    lax.fori_loop(0, ZR, fill_z, 0)

    def fill_o(r, _):
      obuf[r, :] = jnp.ones((D_H,), jnp.float32)
      return 0
    lax.fori_loop(0, CB, fill_o, 0)

    for k in range(RPS // ZR):
      pltpu.sync_copy(zbuf, acc.at[pl.ds(s * RPS + k * ZR, ZR)])
    plsc.subcore_barrier()

    def chunk(i, _):
      pltpu.sync_copy(obuf, acc.at[dst_v.at[i]], add=True)
      return 0
    lax.fori_loop(0, NCHUNK, chunk, 0)

    plsc.subcore_barrier()
    pltpu.sync_copy(acc.at[pl.ds(s * RPS, RPS)],
                    out_hbm.at[c, pl.ds(s * RPS, RPS)])

  return deg_kernel


def _make_agg(d):
  scratch = [
      pltpu.VMEM((NCHUNK, CB), jnp.int32),
      pltpu.VMEM((NCHUNK, CB), jnp.int32),
      pltpu.VMEM((CB, d), jnp.float32),
      pltpu.VMEM((ZR, d), jnp.float32),
      pltpu.VMEM_SHARED((ACC_N, d), jnp.float32),
      pltpu.SemaphoreType.DMA,
  ]

  @functools.partial(
      pl.kernel,
      out_type=jax.ShapeDtypeStruct((NC, ACC_N, d), jnp.float32),
      mesh=_MESH,
      scratch_types=scratch,
      compiler_params=pltpu.CompilerParams(use_tc_tiling_on_sc=False),
  )
  def agg_kernel(y_hbm, ei_hbm, out_hbm, src_v, dst_v, buf, zbuf, acc, sem):
    c = lax.axis_index("c")
    s = lax.axis_index("s")
    wid = c * NS + s
    pltpu.sync_copy(ei_hbm.at[0, wid], src_v)
    pltpu.sync_copy(ei_hbm.at[1, wid], dst_v)

    def fill_z(r, _):
      for j in range(d // 16):
        zbuf[r, pl.ds(j * 16, 16)] = jnp.zeros((16,), jnp.float32)
      return 0
    lax.fori_loop(0, ZR, fill_z, 0)

    for k in range(RPS // ZR):
      pltpu.sync_copy(zbuf, acc.at[pl.ds(s * RPS + k * ZR, ZR)])
    plsc.subcore_barrier()

    def chunk(i, _):
      pltpu.async_copy(y_hbm.at[src_v.at[i]], buf, sem).wait()
      pltpu.sync_copy(buf, acc.at[dst_v.at[i]], add=True)
      return 0
    lax.fori_loop(0, NCHUNK, chunk, 0)

    plsc.subcore_barrier()
    pltpu.sync_copy(acc.at[pl.ds(s * RPS, RPS)],
                    out_hbm.at[c, pl.ds(s * RPS, RPS)])

  return agg_kernel


_deg_call = _make_deg()
_agg16_call = _make_agg(D_H)
_agg64_call = _make_agg(D_OUT)


BN = 2000


def _layer1_body(x_ref, w1_ref, dg_ref, y1_ref, dinv_ref):
  deg = dg_ref[0] + dg_ref[1] + 1.0
  dinv = lax.rsqrt(deg)
  xw = jnp.dot(x_ref[...], w1_ref[...], preferred_element_type=jnp.float32)
  y1_ref[...] = xw * dinv
  dinv_ref[...] = dinv


def _layer1_call(x, w1, degp):
  return pl.pallas_call(
      _layer1_body,
      grid=(N // BN,),
      in_specs=[
          pl.BlockSpec((BN, D_IN), lambda i: (i, 0)),
          pl.BlockSpec((D_IN, D_H), lambda i: (0, 0)),
          pl.BlockSpec((NC, BN, D_H), lambda i: (0, i, 0)),
      ],
      out_specs=[
          pl.BlockSpec((BN, D_H), lambda i: (i, 0)),
          pl.BlockSpec((BN, D_H), lambda i: (i, 0)),
      ],
      out_shape=[
          jax.ShapeDtypeStruct((N, D_H), jnp.float32),
          jax.ShapeDtypeStruct((N, D_H), jnp.float32),
      ],
  )(x, w1, degp)


def _layer2_body(ag_ref, y1_ref, dinv_ref, b1_ref, w2_ref, y2_ref):
  t = ag_ref[0] + ag_ref[1] + y1_ref[...]
  h = jnp.maximum(dinv_ref[...] * t + b1_ref[...], 0.0)
  hw = jnp.dot(h, w2_ref[...], preferred_element_type=jnp.float32)
  dinv64 = lax.broadcast_in_dim(dinv_ref[...][:, 0:1], (BN, D_OUT), (0, 1))
  y2_ref[...] = hw * dinv64


def _layer2_call(agg1, y1, dinv, b1r, w2):
  return pl.pallas_call(
      _layer2_body,
      grid=(N // BN,),
      in_specs=[
          pl.BlockSpec((NC, BN, D_H), lambda i: (0, i, 0)),
          pl.BlockSpec((BN, D_H), lambda i: (i, 0)),
          pl.BlockSpec((BN, D_H), lambda i: (i, 0)),
          pl.BlockSpec((1, D_H), lambda i: (0, 0)),
          pl.BlockSpec((D_H, D_OUT), lambda i: (0, 0)),
      ],
      out_specs=pl.BlockSpec((BN, D_OUT), lambda i: (i, 0)),
      out_shape=jax.ShapeDtypeStruct((N, D_OUT), jnp.float32),
  )(agg1, y1, dinv, b1r, w2)


def _out_body(ag_ref, y2_ref, dinv_ref, b2_ref, o_ref):
  dinv64 = lax.broadcast_in_dim(dinv_ref[...][:, 0:1], (BN, D_OUT), (0, 1))
  o = dinv64 * (ag_ref[0] + ag_ref[1] + y2_ref[...]) + b2_ref[...]
  m = jnp.max(o, axis=1, keepdims=True)
  lse = jnp.log(jnp.sum(jnp.exp(o - m), axis=1, keepdims=True)) + m
  o_ref[...] = o - lse


def _out_call(agg2, y2, dinv, b2r):
  return pl.pallas_call(
      _out_body,
      grid=(N // BN,),
      in_specs=[
          pl.BlockSpec((NC, BN, D_OUT), lambda i: (0, i, 0)),
          pl.BlockSpec((BN, D_OUT), lambda i: (i, 0)),
          pl.BlockSpec((BN, D_H), lambda i: (i, 0)),
          pl.BlockSpec((1, D_OUT), lambda i: (0, 0)),
      ],
      out_specs=pl.BlockSpec((BN, D_OUT), lambda i: (i, 0)),
      out_shape=jax.ShapeDtypeStruct((N, D_OUT), jnp.float32),
  )(agg2, y2, dinv, b2r)


def kernel(x, edge_index, W1, b1, W2, b2):
  ei = edge_index.astype(jnp.int32).reshape(2, NW, NCHUNK, CB)
  degp = _deg_call(ei)
  y1, dinv = _layer1_call(x, W1, degp)
  agg1 = _agg16_call(y1, ei)
  y2 = _layer2_call(agg1, y1, dinv, b1.reshape(1, D_H), W2)
  agg2 = _agg64_call(y2, ei)
  return _out_call(agg2, y2, dinv, b2.reshape(1, D_OUT))


# speedup vs baseline: 39.6694x; 1.4505x over previous
import functools

import jax
import jax.numpy as jnp
from jax import lax
from jax.experimental import pallas as pl
from jax.experimental.pallas import tpu as pltpu
from jax.experimental.pallas import tpu_sc as plsc

N = 10000
E = 320000
D_IN = 128
D_H = 16
D_OUT = 64

NC = 2
NS = 16
NW = NC * NS
EPT = E // NW
CB = 80
NCHUNK = EPT // CB
ACC_N = 10240
RPS = ACC_N // NS
ZR = 128

_MESH = plsc.VectorSubcoreMesh(
    core_axis_name="c", subcore_axis_name="s", num_cores=NC, num_subcores=NS)


def _make_deg():
  scratch = [
      pltpu.VMEM((NCHUNK, CB), jnp.int32),
      pltpu.VMEM((ZR, D_H), jnp.float32),
      pltpu.VMEM((CB, D_H), jnp.float32),
      pltpu.VMEM_SHARED((ACC_N, D_H), jnp.float32),
  ]

  @functools.partial(
      pl.kernel,
      out_type=jax.ShapeDtypeStruct((NC, ACC_N, D_H), jnp.float32),
      mesh=_MESH,
      scratch_types=scratch,
      compiler_params=pltpu.CompilerParams(use_tc_tiling_on_sc=False),
  )
  def deg_kernel(ei_hbm, out_hbm, dst_v, zbuf, obuf, acc):
    c = lax.axis_index("c")
    s = lax.axis_index("s")
    wid = c * NS + s
    pltpu.sync_copy(ei_hbm.at[1, wid], dst_v)

    def fill_z(r, _):
      zbuf[r, :] = jnp.zeros((D_H,), jnp.float32)
      return 0
    lax.fori_loop(0, ZR, fill_z, 0)

    def fill_o(r, _):
      obuf[r, :] = jnp.ones((D_H,), jnp.float32)
      return 0
    lax.fori_loop(0, CB, fill_o, 0)

    for k in range(RPS // ZR):
      pltpu.sync_copy(zbuf, acc.at[pl.ds(s * RPS + k * ZR, ZR)])
    plsc.subcore_barrier()

    def chunk(i, _):
      pltpu.sync_copy(obuf, acc.at[dst_v.at[i]], add=True)
      return 0
    lax.fori_loop(0, NCHUNK, chunk, 0)

    plsc.subcore_barrier()
    pltpu.sync_copy(acc.at[pl.ds(s * RPS, RPS)],
                    out_hbm.at[c, pl.ds(s * RPS, RPS)])

  return deg_kernel


def _make_agg(d):
  scratch = [
      pltpu.VMEM((NCHUNK, CB), jnp.int32),
      pltpu.VMEM((NCHUNK, CB), jnp.int32),
      pltpu.VMEM((CB, d), jnp.float32),
      pltpu.VMEM((CB, d), jnp.float32),
      pltpu.VMEM((ZR, d), jnp.float32),
      pltpu.VMEM_SHARED((ACC_N, d), jnp.float32),
      pltpu.SemaphoreType.DMA,
      pltpu.SemaphoreType.DMA,
  ]

  @functools.partial(
      pl.kernel,
      out_type=jax.ShapeDtypeStruct((NC, ACC_N, d), jnp.float32),
      mesh=_MESH,
      scratch_types=scratch,
      compiler_params=pltpu.CompilerParams(use_tc_tiling_on_sc=False),
  )
  def agg_kernel(y_hbm, ei_hbm, out_hbm, src_v, dst_v, buf0, buf1, zbuf, acc,
                 sem0, sem1):
    c = lax.axis_index("c")
    s = lax.axis_index("s")
    wid = c * NS + s
    pltpu.sync_copy(ei_hbm.at[0, wid], src_v)
    pltpu.sync_copy(ei_hbm.at[1, wid], dst_v)

    def fill_z(r, _):
      for j in range(d // 16):
        zbuf[r, pl.ds(j * 16, 16)] = jnp.zeros((16,), jnp.float32)
      return 0
    lax.fori_loop(0, ZR, fill_z, 0)

    for k in range(RPS // ZR):
      pltpu.sync_copy(zbuf, acc.at[pl.ds(s * RPS + k * ZR, ZR)])
    plsc.subcore_barrier()

    pltpu.async_copy(y_hbm.at[src_v.at[0]], buf0, sem0)

    def pair(j, _):
      i0 = 2 * j
      pltpu.async_copy(y_hbm.at[src_v.at[i0 + 1]], buf1, sem1)
      pltpu.make_async_copy(y_hbm.at[src_v.at[i0]], buf0, sem0).wait()
      pltpu.sync_copy(buf0, acc.at[dst_v.at[i0]], add=True)
      pltpu.async_copy(y_hbm.at[src_v.at[i0 + 2]], buf0, sem0)
      pltpu.make_async_copy(y_hbm.at[src_v.at[i0 + 1]], buf1, sem1).wait()
      pltpu.sync_copy(buf1, acc.at[dst_v.at[i0 + 1]], add=True)
      return 0
    lax.fori_loop(0, (NCHUNK - 1) // 2, pair, 0)

    pltpu.make_async_copy(y_hbm.at[src_v.at[NCHUNK - 1]], buf0, sem0).wait()
    pltpu.sync_copy(buf0, acc.at[dst_v.at[NCHUNK - 1]], add=True)

    plsc.subcore_barrier()
    pltpu.sync_copy(acc.at[pl.ds(s * RPS, RPS)],
                    out_hbm.at[c, pl.ds(s * RPS, RPS)])

  return agg_kernel


_deg_call = _make_deg()
_agg16_call = _make_agg(D_H)
_agg64_call = _make_agg(D_OUT)


BN = 2000


def _layer1_body(x_ref, w1_ref, dg_ref, y1_ref, dinv_ref):
  deg = dg_ref[0] + dg_ref[1] + 1.0
  dinv = lax.rsqrt(deg)
  xw = jnp.dot(x_ref[...], w1_ref[...], preferred_element_type=jnp.float32)
  y1_ref[...] = xw * dinv
  dinv_ref[...] = dinv


def _layer1_call(x, w1, degp):
  return pl.pallas_call(
      _layer1_body,
      grid=(N // BN,),
      in_specs=[
          pl.BlockSpec((BN, D_IN), lambda i: (i, 0)),
          pl.BlockSpec((D_IN, D_H), lambda i: (0, 0)),
          pl.BlockSpec((NC, BN, D_H), lambda i: (0, i, 0)),
      ],
      out_specs=[
          pl.BlockSpec((BN, D_H), lambda i: (i, 0)),
          pl.BlockSpec((BN, D_H), lambda i: (i, 0)),
      ],
      out_shape=[
          jax.ShapeDtypeStruct((N, D_H), jnp.float32),
          jax.ShapeDtypeStruct((N, D_H), jnp.float32),
      ],
  )(x, w1, degp)


def _layer2_body(ag_ref, y1_ref, dinv_ref, b1_ref, w2_ref, y2_ref):
  t = ag_ref[0] + ag_ref[1] + y1_ref[...]
  h = jnp.maximum(dinv_ref[...] * t + b1_ref[...], 0.0)
  hw = jnp.dot(h, w2_ref[...], preferred_element_type=jnp.float32)
  dinv64 = lax.broadcast_in_dim(dinv_ref[...][:, 0:1], (BN, D_OUT), (0, 1))
  y2_ref[...] = hw * dinv64


def _layer2_call(agg1, y1, dinv, b1r, w2):
  return pl.pallas_call(
      _layer2_body,
      grid=(N // BN,),
      in_specs=[
          pl.BlockSpec((NC, BN, D_H), lambda i: (0, i, 0)),
          pl.BlockSpec((BN, D_H), lambda i: (i, 0)),
          pl.BlockSpec((BN, D_H), lambda i: (i, 0)),
          pl.BlockSpec((1, D_H), lambda i: (0, 0)),
          pl.BlockSpec((D_H, D_OUT), lambda i: (0, 0)),
      ],
      out_specs=pl.BlockSpec((BN, D_OUT), lambda i: (i, 0)),
      out_shape=jax.ShapeDtypeStruct((N, D_OUT), jnp.float32),
  )(agg1, y1, dinv, b1r, w2)


def _out_body(ag_ref, y2_ref, dinv_ref, b2_ref, o_ref):
  dinv64 = lax.broadcast_in_dim(dinv_ref[...][:, 0:1], (BN, D_OUT), (0, 1))
  o = dinv64 * (ag_ref[0] + ag_ref[1] + y2_ref[...]) + b2_ref[...]
  m = jnp.max(o, axis=1, keepdims=True)
  lse = jnp.log(jnp.sum(jnp.exp(o - m), axis=1, keepdims=True)) + m
  o_ref[...] = o - lse


def _out_call(agg2, y2, dinv, b2r):
  return pl.pallas_call(
      _out_body,
      grid=(N // BN,),
      in_specs=[
          pl.BlockSpec((NC, BN, D_OUT), lambda i: (0, i, 0)),
          pl.BlockSpec((BN, D_OUT), lambda i: (i, 0)),
          pl.BlockSpec((BN, D_H), lambda i: (i, 0)),
          pl.BlockSpec((1, D_OUT), lambda i: (0, 0)),
      ],
      out_specs=pl.BlockSpec((BN, D_OUT), lambda i: (i, 0)),
      out_shape=jax.ShapeDtypeStruct((N, D_OUT), jnp.float32),
  )(agg2, y2, dinv, b2r)


def kernel(x, edge_index, W1, b1, W2, b2):
  ei = edge_index.astype(jnp.int32).reshape(2, NW, NCHUNK, CB)
  degp = _deg_call(ei)
  y1, dinv = _layer1_call(x, W1, degp)
  agg1 = _agg16_call(y1, ei)
  y2 = _layer2_call(agg1, y1, dinv, b1.reshape(1, D_H), W2)
  agg2 = _agg64_call(y2, ei)
  return _out_call(agg2, y2, dinv, b2.reshape(1, D_OUT))
